# baseline (speedup 1.0000x reference)
import functools

import jax
import jax.numpy as jnp
from jax import lax
from jax.experimental import pallas as pl
from jax.experimental.pallas import tpu as pltpu

(_G, _STD_IN, _STD_OUT, _ALPHA, _RHO, _K1, _K2, _K3,
 _V, _E0, _TAU_S, _TAU_F, _TAU_0, _DT, _SQRT_DT) = range(15)
_NUM_PARAMS = 16

_U32 = jnp.uint32
_LO = float(-0.99999994)
_SPAN = 2.0
_SQRT2 = 1.4142135623730951

_ROT_A = (13, 15, 26, 6)
_ROT_B = (17, 29, 16, 24)


def _rotl(x, d):
    return (x << _U32(d)) | (x >> _U32(32 - d))


def _threefry_bits(k0, k1, idx):
    ks2 = k0 ^ k1 ^ _U32(0x1BD11BDA)
    x0 = jnp.zeros_like(idx) + k0
    x1 = idx + k1
    ks = (k0, k1, ks2)
    for g in range(5):
        rots = _ROT_A if g % 2 == 0 else _ROT_B
        for r in rots:
            x0 = x0 + x1
            x1 = _rotl(x1, r) ^ x0
        x0 = x0 + ks[(g + 1) % 3]
        x1 = x1 + ks[(g + 2) % 3] + _U32(g + 1)
    return x0 ^ x1


def _erfinv(x):
    w = -jnp.log1p(-x * x)
    w_lt = w < 5.0
    wa = w - 2.5
    wb = jnp.sqrt(jnp.maximum(w, 5.0)) - 3.0
    pa = 2.81022636e-08
    pb = -0.000200214257
    for a, b in zip(
        (3.43273939e-07, -3.5233877e-06, -4.39150654e-06, 0.00021858087,
         -0.00125372503, -0.00417768164, 0.246640727, 1.50140941),
        (0.000100950558, 0.00134934322, -0.00367342844, 0.00573950773,
         -0.0076224613, 0.00943887047, 1.00167406, 2.83297682)):
        pa = a + pa * wa
        pb = b + pb * wb
    return jnp.where(w_lt, pa, pb) * x


def _normal_from_idx(k0, k1, idx):
    bits = _threefry_bits(k0, k1, idx)
    fb = (bits >> _U32(9)) | _U32(0x3F800000)
    f1 = lax.bitcast_convert_type(fb, jnp.float32) - 1.0
    u = f1 * _SPAN + _LO
    return _SQRT2 * _erfinv(u)


def _lap_kernel(params_ref, sc_ref, gc_ref, lap_ref):
    g = params_ref[_G]
    sc_mod = jnp.exp(gc_ref[...]) * sc_ref[...]
    sc_sym = 0.5 * (sc_mod + sc_mod.T)
    fro = jnp.sqrt(jnp.sum(sc_sym * sc_sym))
    sc_n = sc_sym / fro
    row_sum = jnp.sum(sc_n, axis=1, keepdims=True)
    rr = lax.broadcasted_iota(jnp.int32, sc_n.shape, 0)
    cc = lax.broadcasted_iota(jnp.int32, sc_n.shape, 1)
    lap = jnp.where(rr == cc, sc_n - row_sum, sc_n)
    lap_ref[...] = g * lap


def _sim_kernel(params_ref, keys_ref, lap_ref, hx_ref, ext_ref,
                swin_ref, bold_ref, cur_ref, st_ref,
                *, steps_per_tr, trs_per_window, b_block, n_nodes, batch):
    t = pl.program_id(1)
    b = pl.program_id(0)

    std_in  = params_ref[_STD_IN]
    std_out = params_ref[_STD_OUT]
    alpha   = params_ref[_ALPHA]
    rho     = params_ref[_RHO]
    k1      = params_ref[_K1]
    k2      = params_ref[_K2]
    k3      = params_ref[_K3]
    V       = params_ref[_V]
    E0      = params_ref[_E0]
    tau_s   = params_ref[_TAU_S]
    tau_f   = params_ref[_TAU_F]
    tau_0   = params_ref[_TAU_0]
    dt      = params_ref[_DT]
    sqrt_dt = params_ref[_SQRT_DT]

    ke0 = keys_ref[0]
    ke1 = keys_ref[1]
    kb0 = keys_ref[2]
    kb1 = keys_ref[3]

    inv_alpha    = 1.0 / alpha
    inv_alpha_m1 = inv_alpha - 1.0
    inv_rho      = 1.0 / rho
    inv_tau_s    = 1.0 / tau_s
    inv_tau_f    = 1.0 / tau_f
    dt_tau0      = dt / tau_0
    log1m_rho    = jnp.log(1.0 - rho)
    noise_scale  = sqrt_dt * (0.1 + std_in)
    bold_gain    = 100.0 * V / E0

    @pl.when(t == 0)
    def _init():
        st_ref[...] = hx_ref[...]

    E = st_ref[0]
    x = st_ref[1]
    f = st_ref[2]
    v = st_ref[3]
    q = st_ref[4]
    lap_g = lap_ref[...]

    off2d = (lax.broadcasted_iota(_U32, (b_block, n_nodes), 0) * _U32(n_nodes)
             + lax.broadcasted_iota(_U32, (b_block, n_nodes), 1))
    row0 = (b * b_block).astype(_U32) * _U32(n_nodes)
    plane = _U32(batch * n_nodes)

    for s in range(steps_per_tr):
        u = ext_ref[0, s:s + 1, :]
        step = t * steps_per_tr + s
        idx_e = step.astype(_U32) * plane + row0 + off2d
        nE = _normal_from_idx(ke0, ke1, idx_e)

        IE = jnp.dot(E, lap_g, preferred_element_type=jnp.float32) + u

        lv        = jnp.log(v)
        v_pow     = jnp.exp(inv_alpha * lv)
        v_pow_dv  = jnp.exp(inv_alpha_m1 * lv)
        pow_rho_f = jnp.exp(log1m_rho * pl.reciprocal(f, approx=True))

        E_next = E + dt * (-E + jnp.tanh(IE)) + noise_scale * nE
        x_next = x + dt * (E - x * inv_tau_s - (f - 1.0) * inv_tau_f)
        f_next = f + dt * x
        v_next = v + dt_tau0 * (f - v_pow)
        q_next = q + dt_tau0 * (f * (1.0 - pow_rho_f) * inv_rho - q * v_pow_dv)

        E = jnp.tanh(E_next)
        x = x_next
        f = 1.0 + jnp.tanh(f_next - 1.0)
        v = 1.0 + jnp.tanh(v_next - 1.0)
        q = 1.0 + jnp.tanh(q_next - 1.0)

    st_ref[0] = E
    st_ref[1] = x
    st_ref[2] = f
    st_ref[3] = v
    st_ref[4] = q

    swin_ref[0, 0] = E
    swin_ref[1, 0] = x
    swin_ref[2, 0] = f
    swin_ref[3, 0] = v
    swin_ref[4, 0] = q

    idx_b = t.astype(_U32) * plane + row0 + off2d
    nB = _normal_from_idx(kb0, kb1, idx_b)
    bold_ref[0] = (std_out * nB
                   + bold_gain * (k1 * (1.0 - q)
                                  + k2 * (1.0 - q / v)
                                  + k3 * (1.0 - v)))

    @pl.when(t == trs_per_window - 1)
    def _finalize():
        cur_ref[...] = st_ref[...]


def kernel(external, hx_batch, hE, sc, gains_con, g, std_in, std_out, alpha,
           rho, k1, k2, k3, V, E0, tau_s, tau_f, tau_0, noise_seed):
    step_size = 0.05
    tr = 0.75
    S = int(tr / step_size)
    N, _, T = external.shape
    B = hx_batch.shape[0]

    nb = 1
    b_block = B // nb

    p = jnp.zeros((_NUM_PARAMS,), jnp.float32)
    p = p.at[:15].set(jnp.array(
        [g, std_in, std_out, alpha, rho, k1, k2, k3, V, E0,
         tau_s, tau_f, tau_0, step_size, step_size ** 0.5], jnp.float32))

    noise_key = jax.random.wrap_key_data(noise_seed)
    k_e, k_b = jax.random.split(noise_key)
    keys = jnp.concatenate([jax.random.key_data(k_e),
                            jax.random.key_data(k_b)]).astype(jnp.uint32)

    ext = jnp.transpose(external.astype(jnp.float32), (2, 1, 0))
    hx_sbn = jnp.transpose(hx_batch.astype(jnp.float32), (2, 0, 1))
    sc32 = sc.astype(jnp.float32)
    gc32 = gains_con.astype(jnp.float32)

    lap_g = pl.pallas_call(
        _lap_kernel,
        out_shape=jax.ShapeDtypeStruct((N, N), jnp.float32),
        grid=(1,),
        in_specs=[
            pl.BlockSpec((_NUM_PARAMS,), lambda i: (0,),
                         memory_space=pltpu.MemorySpace.SMEM),
            pl.BlockSpec((N, N), lambda i: (0, 0)),
            pl.BlockSpec((N, N), lambda i: (0, 0)),
        ],
        out_specs=pl.BlockSpec((N, N), lambda i: (0, 0)),
    )(p, sc32, gc32)

    _kernel_fn = functools.partial(_sim_kernel,
                                   steps_per_tr=S, trs_per_window=T,
                                   b_block=b_block, n_nodes=N, batch=B)

    in_specs = [
        pl.BlockSpec((_NUM_PARAMS,), lambda b, t: (0,),
                     memory_space=pltpu.MemorySpace.SMEM),
        pl.BlockSpec((4,), lambda b, t: (0,),
                     memory_space=pltpu.MemorySpace.SMEM),
        pl.BlockSpec((N, N), lambda b, t: (0, 0)),
        pl.BlockSpec((5, b_block, N), lambda b, t: (0, b, 0)),
        pl.BlockSpec((1, S, N), lambda b, t: (t, 0, 0)),
    ]
    out_specs = [
        pl.BlockSpec((5, 1, b_block, N), lambda b, t: (0, t, b, 0)),
        pl.BlockSpec((1, b_block, N), lambda b, t: (t, b, 0)),
        pl.BlockSpec((5, b_block, N), lambda b, t: (0, b, 0)),
    ]
    out_shapes = (
        jax.ShapeDtypeStruct((5, T, B, N), jnp.float32),
        jax.ShapeDtypeStruct((T, B, N), jnp.float32),
        jax.ShapeDtypeStruct((5, B, N), jnp.float32),
    )

    state_win, bold_win, cur = pl.pallas_call(
        _kernel_fn,
        out_shape=out_shapes,
        grid=(nb, T),
        in_specs=in_specs,
        out_specs=out_specs,
        scratch_shapes=[pltpu.VMEM((5, b_block, N), jnp.float32)],
        compiler_params=pltpu.CompilerParams(
            dimension_semantics=("parallel", "arbitrary")),
    )(p, keys, lap_g, hx_sbn, ext)

    next_state = {
        "current_state": jnp.transpose(cur, (1, 2, 0)),
        "bold_window":   jnp.transpose(bold_win, (1, 2, 0)),
        "E_window":      jnp.transpose(state_win[0], (1, 2, 0)),
        "x_window":      jnp.transpose(state_win[1], (1, 2, 0)),
        "f_window":      jnp.transpose(state_win[2], (1, 2, 0)),
        "v_window":      jnp.transpose(state_win[3], (1, 2, 0)),
        "q_window":      jnp.transpose(state_win[4], (1, 2, 0)),
    }
    return next_state, hE


# speedup vs baseline: 1.3997x; 1.3997x over previous
import functools

import jax
import jax.numpy as jnp
from jax import lax
from jax.experimental import pallas as pl
from jax.experimental.pallas import tpu as pltpu

(_G, _STD_IN, _STD_OUT, _ALPHA, _RHO, _K1, _K2, _K3,
 _V, _E0, _TAU_S, _TAU_F, _TAU_0, _DT, _SQRT_DT) = range(15)
_NUM_PARAMS = 16


def _lap_kernel(params_ref, sc_ref, gc_ref, lap_ref):
    g = params_ref[_G]
    sc_mod = jnp.exp(gc_ref[...]) * sc_ref[...]
    sc_sym = 0.5 * (sc_mod + sc_mod.T)
    fro = jnp.sqrt(jnp.sum(sc_sym * sc_sym))
    sc_n = sc_sym / fro
    row_sum = jnp.sum(sc_n, axis=1, keepdims=True)
    rr = lax.broadcasted_iota(jnp.int32, sc_n.shape, 0)
    cc = lax.broadcasted_iota(jnp.int32, sc_n.shape, 1)
    lap = jnp.where(rr == cc, sc_n - row_sum, sc_n)
    lap_ref[...] = g * lap


def _sim_kernel(params_ref, lap_ref, hx_ref, ext_ref, ne_ref, nb_ref,
                swin_ref, bold_ref, cur_ref, st_ref,
                *, steps_per_tr, trs_per_window):
    t = pl.program_id(1)

    std_in  = params_ref[_STD_IN]
    std_out = params_ref[_STD_OUT]
    alpha   = params_ref[_ALPHA]
    rho     = params_ref[_RHO]
    k1      = params_ref[_K1]
    k2      = params_ref[_K2]
    k3      = params_ref[_K3]
    V       = params_ref[_V]
    E0      = params_ref[_E0]
    tau_s   = params_ref[_TAU_S]
    tau_f   = params_ref[_TAU_F]
    tau_0   = params_ref[_TAU_0]
    dt      = params_ref[_DT]
    sqrt_dt = params_ref[_SQRT_DT]

    inv_alpha    = 1.0 / alpha
    inv_alpha_m1 = inv_alpha - 1.0
    inv_rho      = 1.0 / rho
    inv_tau_s    = 1.0 / tau_s
    inv_tau_f    = 1.0 / tau_f
    dt_tau0      = dt / tau_0
    log1m_rho    = jnp.log(1.0 - rho)
    noise_scale  = sqrt_dt * (0.1 + std_in)
    bold_gain    = 100.0 * V / E0

    @pl.when(t == 0)
    def _init():
        st_ref[...] = hx_ref[...]

    E = st_ref[0]
    x = st_ref[1]
    f = st_ref[2]
    v = st_ref[3]
    q = st_ref[4]
    lap_g = lap_ref[...]

    for s in range(steps_per_tr):
        u  = ext_ref[0, s:s + 1, :]
        nE = ne_ref[0, s]

        IE = jnp.dot(E, lap_g, preferred_element_type=jnp.float32) + u

        lv        = jnp.log(v)
        v_pow     = jnp.exp(inv_alpha * lv)
        v_pow_dv  = jnp.exp(inv_alpha_m1 * lv)
        pow_rho_f = jnp.exp(log1m_rho * pl.reciprocal(f, approx=True))

        E_next = E + dt * (-E + jnp.tanh(IE)) + noise_scale * nE
        x_next = x + dt * (E - x * inv_tau_s - (f - 1.0) * inv_tau_f)
        f_next = f + dt * x
        v_next = v + dt_tau0 * (f - v_pow)
        q_next = q + dt_tau0 * (f * (1.0 - pow_rho_f) * inv_rho - q * v_pow_dv)

        E = jnp.tanh(E_next)
        x = x_next
        f = 1.0 + jnp.tanh(f_next - 1.0)
        v = 1.0 + jnp.tanh(v_next - 1.0)
        q = 1.0 + jnp.tanh(q_next - 1.0)

    st_ref[0] = E
    st_ref[1] = x
    st_ref[2] = f
    st_ref[3] = v
    st_ref[4] = q

    swin_ref[0, 0] = E
    swin_ref[1, 0] = x
    swin_ref[2, 0] = f
    swin_ref[3, 0] = v
    swin_ref[4, 0] = q

    bold_ref[0] = (std_out * nb_ref[0]
                   + bold_gain * (k1 * (1.0 - q)
                                  + k2 * (1.0 - q / v)
                                  + k3 * (1.0 - v)))

    @pl.when(t == trs_per_window - 1)
    def _finalize():
        cur_ref[...] = st_ref[...]


def kernel(external, hx_batch, hE, sc, gains_con, g, std_in, std_out, alpha,
           rho, k1, k2, k3, V, E0, tau_s, tau_f, tau_0, noise_seed):
    step_size = 0.05
    tr = 0.75
    S = int(tr / step_size)
    N, _, T = external.shape
    B = hx_batch.shape[0]
    TS = T * S

    nb = 1
    b_block = B // nb

    p = jnp.zeros((_NUM_PARAMS,), jnp.float32)
    p = p.at[:15].set(jnp.array(
        [g, std_in, std_out, alpha, rho, k1, k2, k3, V, E0,
         tau_s, tau_f, tau_0, step_size, step_size ** 0.5], jnp.float32))

    noise_key = jax.random.wrap_key_data(noise_seed)
    k_e, k_b = jax.random.split(noise_key)
    noise_e = jax.random.normal(k_e, (TS, B, N), jnp.float32).reshape(T, S, B, N)
    noise_b = jax.random.normal(k_b, (T, B, N), jnp.float32)

    ext = jnp.transpose(external.astype(jnp.float32), (2, 1, 0))
    hx_sbn = jnp.transpose(hx_batch.astype(jnp.float32), (2, 0, 1))
    sc32 = sc.astype(jnp.float32)
    gc32 = gains_con.astype(jnp.float32)

    lap_g = pl.pallas_call(
        _lap_kernel,
        out_shape=jax.ShapeDtypeStruct((N, N), jnp.float32),
        grid=(1,),
        in_specs=[
            pl.BlockSpec((_NUM_PARAMS,), lambda i: (0,),
                         memory_space=pltpu.MemorySpace.SMEM),
            pl.BlockSpec((N, N), lambda i: (0, 0)),
            pl.BlockSpec((N, N), lambda i: (0, 0)),
        ],
        out_specs=pl.BlockSpec((N, N), lambda i: (0, 0)),
    )(p, sc32, gc32)

    _kernel_fn = functools.partial(_sim_kernel,
                                   steps_per_tr=S, trs_per_window=T)

    in_specs = [
        pl.BlockSpec((_NUM_PARAMS,), lambda b, t: (0,),
                     memory_space=pltpu.MemorySpace.SMEM),
        pl.BlockSpec((N, N), lambda b, t: (0, 0)),
        pl.BlockSpec((5, b_block, N), lambda b, t: (0, b, 0)),
        pl.BlockSpec((1, S, N), lambda b, t: (t, 0, 0)),
        pl.BlockSpec((1, S, b_block, N), lambda b, t: (t, 0, b, 0)),
        pl.BlockSpec((1, b_block, N), lambda b, t: (t, b, 0)),
    ]
    out_specs = [
        pl.BlockSpec((5, 1, b_block, N), lambda b, t: (0, t, b, 0)),
        pl.BlockSpec((1, b_block, N), lambda b, t: (t, b, 0)),
        pl.BlockSpec((5, b_block, N), lambda b, t: (0, b, 0)),
    ]
    out_shapes = (
        jax.ShapeDtypeStruct((5, T, B, N), jnp.float32),
        jax.ShapeDtypeStruct((T, B, N), jnp.float32),
        jax.ShapeDtypeStruct((5, B, N), jnp.float32),
    )

    state_win, bold_win, cur = pl.pallas_call(
        _kernel_fn,
        out_shape=out_shapes,
        grid=(nb, T),
        in_specs=in_specs,
        out_specs=out_specs,
        scratch_shapes=[pltpu.VMEM((5, b_block, N), jnp.float32)],
        compiler_params=pltpu.CompilerParams(
            dimension_semantics=("parallel", "arbitrary")),
    )(p, lap_g, hx_sbn, ext, noise_e, noise_b)

    next_state = {
        "current_state": jnp.transpose(cur, (1, 2, 0)),
        "bold_window":   jnp.transpose(bold_win, (1, 2, 0)),
        "E_window":      jnp.transpose(state_win[0], (1, 2, 0)),
        "x_window":      jnp.transpose(state_win[1], (1, 2, 0)),
        "f_window":      jnp.transpose(state_win[2], (1, 2, 0)),
        "v_window":      jnp.transpose(state_win[3], (1, 2, 0)),
        "q_window":      jnp.transpose(state_win[4], (1, 2, 0)),
    }
    return next_state, hE
